# Initial kernel scaffold; baseline (speedup 1.0000x reference)
import jax
import jax.numpy as jnp
from jax.experimental import pallas as pl


def kernel(x, adj, adj_high, W0, b0, W1, b1, W2, b2, W_out, b_out):
    raise NotImplementedError("write your pallas kernel here")


# speedup vs baseline: 1.1384x; 1.1384x over previous
import jax
import jax.numpy as jnp
from jax.experimental import pallas as pl
from jax.experimental.pallas import tpu as pltpu

f32 = jnp.float32
bf16 = jnp.bfloat16


def _dot(a, b):
    return jax.lax.dot_general(a, b, (((1,), (0,)), ((), ())),
                               preferred_element_type=f32)


def kernel(x, adj, adj_high, W0, b0, W1, b1, W2, b2, W_out, b_out):
    N, F = x.shape
    H = W0.shape[1]
    C = W_out.shape[1]
    BM1 = 400
    BM2 = 1000

    x_bf = x.astype(bf16)
    W0b = W0.astype(bf16)
    W1x = W1[:F].astype(bf16)
    W1h = W1[F:].astype(bf16)
    Wc_x = jnp.concatenate([W2[:F], W_out[:F]], axis=1).astype(bf16)
    Wc_h0 = jnp.concatenate([W2[F:F + H], W_out[F:F + H]], axis=1).astype(bf16)
    Wc_h1 = jnp.concatenate([W2[F + H:], W_out[F + H:F + 2 * H]], axis=1).astype(bf16)
    Wo2 = W_out[F + 2 * H:].astype(bf16)
    b0r = b0.reshape(1, H).astype(f32)
    b1r = b1.reshape(1, H).astype(f32)
    b2r = b2.reshape(1, H).astype(f32)
    boutr = b_out.reshape(1, C).astype(f32)

    cparams = pltpu.CompilerParams(dimension_semantics=("arbitrary",))

    def b1_body(x_ref, w_ref, o_ref):
        o_ref[...] = _dot(x_ref[...], w_ref[...]).astype(bf16)

    B1 = pl.pallas_call(
        b1_body,
        grid=(N // BM2,),
        in_specs=[pl.BlockSpec((BM2, F), lambda i: (i, 0)),
                  pl.BlockSpec((F, H), lambda i: (0, 0))],
        out_specs=pl.BlockSpec((BM2, H), lambda i: (i, 0)),
        out_shape=jax.ShapeDtypeStruct((N, H), bf16),
    )(x_bf, W0b)

    def p1_body(adj_ref, b1_ref, x_ref, w1x_ref, w1h_ref, b0_ref,
                adjc_ref, h0_ref, b2_ref):
        t = adj_ref[...].astype(bf16)
        adjc_ref[...] = t
        h0 = jnp.maximum(_dot(t, b1_ref[...]) + b0_ref[...], 0.0)
        h0b = h0.astype(bf16)
        h0_ref[...] = h0b
        b2 = _dot(x_ref[...], w1x_ref[...]) + _dot(h0b, w1h_ref[...])
        b2_ref[...] = b2.astype(bf16)

    adj_c, h0, B2 = pl.pallas_call(
        p1_body,
        grid=(N // BM1,),
        in_specs=[pl.BlockSpec((BM1, N), lambda i: (i, 0)),
                  pl.BlockSpec((N, H), lambda i: (0, 0)),
                  pl.BlockSpec((BM1, F), lambda i: (i, 0)),
                  pl.BlockSpec((F, H), lambda i: (0, 0)),
                  pl.BlockSpec((H, H), lambda i: (0, 0)),
                  pl.BlockSpec((1, H), lambda i: (0, 0))],
        out_specs=[pl.BlockSpec((BM1, N), lambda i: (i, 0)),
                   pl.BlockSpec((BM1, H), lambda i: (i, 0)),
                   pl.BlockSpec((BM1, H), lambda i: (i, 0))],
        out_shape=[jax.ShapeDtypeStruct((N, N), bf16),
                   jax.ShapeDtypeStruct((N, H), bf16),
                   jax.ShapeDtypeStruct((N, H), bf16)],
        compiler_params=cparams,
    )(adj, B1, x_bf, W1x, W1h, b0r)

    def p2_body(adjc_ref, b2_ref, x_ref, h0_ref, wcx_ref, wch0_ref, wch1_ref,
                b1_ref, b3_ref):
        h1 = jnp.maximum(_dot(adjc_ref[...], b2_ref[...]) + b1_ref[...], 0.0)
        b3 = (_dot(x_ref[...], wcx_ref[...])
              + _dot(h0_ref[...], wch0_ref[...])
              + _dot(h1.astype(bf16), wch1_ref[...]))
        b3_ref[...] = b3.astype(bf16)

    B3 = pl.pallas_call(
        p2_body,
        grid=(N // BM2,),
        in_specs=[pl.BlockSpec((BM2, N), lambda i: (i, 0)),
                  pl.BlockSpec((N, H), lambda i: (0, 0)),
                  pl.BlockSpec((BM2, F), lambda i: (i, 0)),
                  pl.BlockSpec((BM2, H), lambda i: (i, 0)),
                  pl.BlockSpec((F, H + C), lambda i: (0, 0)),
                  pl.BlockSpec((H, H + C), lambda i: (0, 0)),
                  pl.BlockSpec((H, H + C), lambda i: (0, 0)),
                  pl.BlockSpec((1, H), lambda i: (0, 0))],
        out_specs=pl.BlockSpec((BM2, H + C), lambda i: (i, 0)),
        out_shape=jax.ShapeDtypeStruct((N, H + C), bf16),
        compiler_params=cparams,
    )(adj_c, B2, x_bf, h0, Wc_x, Wc_h0, Wc_h1, b1r)

    def p3_body(adjc_ref, b3_ref, wo2_ref, b2_ref, bout_ref, b4_ref, part_ref):
        acc = _dot(adjc_ref[...], b3_ref[...])
        h2 = jnp.maximum(acc[:, :H] + b2_ref[...], 0.0)
        part_ref[...] = acc[:, H:] + bout_ref[...]
        b4_ref[...] = _dot(h2.astype(bf16), wo2_ref[...]).astype(bf16)

    B4, partial = pl.pallas_call(
        p3_body,
        grid=(N // BM2,),
        in_specs=[pl.BlockSpec((BM2, N), lambda i: (i, 0)),
                  pl.BlockSpec((N, H + C), lambda i: (0, 0)),
                  pl.BlockSpec((H, C), lambda i: (0, 0)),
                  pl.BlockSpec((1, H), lambda i: (0, 0)),
                  pl.BlockSpec((1, C), lambda i: (0, 0))],
        out_specs=[pl.BlockSpec((BM2, C), lambda i: (i, 0)),
                   pl.BlockSpec((BM2, C), lambda i: (i, 0))],
        out_shape=[jax.ShapeDtypeStruct((N, C), bf16),
                   jax.ShapeDtypeStruct((N, C), f32)],
        compiler_params=cparams,
    )(adj_c, B3, Wo2, b2r, boutr)

    def p4_body(adjc_ref, b4_ref, part_ref, out_ref):
        out_ref[...] = _dot(adjc_ref[...], b4_ref[...]) + part_ref[...]

    out = pl.pallas_call(
        p4_body,
        grid=(N // BM2,),
        in_specs=[pl.BlockSpec((BM2, N), lambda i: (i, 0)),
                  pl.BlockSpec((N, C), lambda i: (0, 0)),
                  pl.BlockSpec((BM2, C), lambda i: (i, 0))],
        out_specs=pl.BlockSpec((BM2, C), lambda i: (i, 0)),
        out_shape=jax.ShapeDtypeStruct((N, C), f32),
        compiler_params=cparams,
    )(adj_c, B4, partial)

    return out
